# baseline (speedup 1.0000x reference)
import functools

import jax
import jax.numpy as jnp
from jax import lax
from jax.experimental import pallas as pl
from jax.experimental.pallas import tpu as pltpu
from jax.experimental.pallas import tpu_sc as plsc

N = 10000
E = 320000
D = 128
H = 128
C = 4
NP = 10240

WS = 16

NC = 2
NS = 16
NW = NC * NS
EPW = E // NW
CK = 80
NCHUNK = EPW // CK
RPT = NP // NS

_f32 = jnp.float32

_mesh = plsc.VectorSubcoreMesh(core_axis_name="c", subcore_axis_name="s",
                               num_cores=NC, num_subcores=NS)


def _make_sc_agg(W, m, tiled):
  @functools.partial(
      pl.kernel,
      out_type=jax.ShapeDtypeStruct((NC, NP, W), _f32),
      mesh=_mesh,
      scratch_types=[
          pltpu.VMEM((EPW,), jnp.int32),
          [pltpu.VMEM((CK,), jnp.int32)] * m,
          [pltpu.VMEM((CK, W), _f32)] * m,
          [pltpu.SemaphoreType.DMA] * m,
          [pltpu.SemaphoreType.DMA] * m,
          pltpu.VMEM_SHARED((NP, W), _f32),
      ],
      compiler_params=pltpu.CompilerParams(use_tc_tiling_on_sc=tiled),
  )
  def agg(z_hbm, src_hbm, dst_hbm, zero_hbm, out_hbm,
          sidx, didx, rows, gsems, dsems, acc):
    c = lax.axis_index("c")
    s = lax.axis_index("s")
    pltpu.sync_copy(zero_hbm, acc.at[pl.ds(s * RPT, RPT)])
    base = (s * NC + c) * EPW
    pltpu.sync_copy(src_hbm.at[pl.ds(base, EPW)], sidx)
    plsc.subcore_barrier()

    def start(j, b):
      pltpu.async_copy(dst_hbm.at[pl.ds(base + j * CK, CK)], didx[b], dsems[b])
      pltpu.async_copy(z_hbm.at[sidx.at[pl.ds(j * CK, CK)]], rows[b], gsems[b])

    def finish(j, b):
      pltpu.make_async_copy(dst_hbm.at[pl.ds(base, CK)], didx[b],
                            dsems[b]).wait()
      pltpu.make_async_copy(z_hbm.at[sidx.at[pl.ds(j * CK, CK)]], rows[b],
                            gsems[b]).wait()
      pltpu.sync_copy(rows[b], acc.at[didx[b]], add=True)

    for b in range(m):
      start(b, b)

    main_iters = NCHUNK // m

    def body(t, carry):
      for b in range(m):
        j = t * m + b
        finish(j, b)

        @pl.when(j + m < NCHUNK)
        def _():
          start(j + m, b)

      return carry

    lax.fori_loop(0, main_iters, body, 0)
    for j in range(main_iters * m, NCHUNK):
      finish(j, j % m)
    plsc.subcore_barrier()
    pltpu.sync_copy(acc.at[pl.ds(s * RPT, RPT)],
                    out_hbm.at[c, pl.ds(s * RPT, RPT)])

  return agg


WH = H // NC
EPT = E // NS
NCH2 = EPT // CK
_M_CS = 5
NRT = N // NS


@functools.partial(
    pl.kernel,
    out_type=jax.ShapeDtypeStruct((NP, H), _f32),
    mesh=_mesh,
    scratch_types=[
        pltpu.VMEM((EPT,), jnp.int32),
        [pltpu.VMEM((CK,), jnp.int32)] * _M_CS,
        [pltpu.VMEM((CK, WH), _f32)] * _M_CS,
        [pltpu.SemaphoreType.DMA] * _M_CS,
        [pltpu.SemaphoreType.DMA] * _M_CS,
        pltpu.VMEM_SHARED((N, WH), _f32),
        pltpu.VMEM_SHARED((NP, WH), _f32),
    ],
    compiler_params=pltpu.CompilerParams(use_tc_tiling_on_sc=False),
)
def _sc_agg_colsplit(z_hbm, src_hbm, dst_hbm, zero_hbm, out_hbm,
                     sidx, didx, rows, gsems, dsems, table, acc):
  c = lax.axis_index("c")
  s = lax.axis_index("s")
  col0 = c * WH
  pltpu.sync_copy(z_hbm.at[pl.ds(s * NRT, NRT), pl.ds(col0, WH)],
                  table.at[pl.ds(s * NRT, NRT)])
  pltpu.sync_copy(zero_hbm, acc.at[pl.ds(s * RPT, RPT)])
  base = s * EPT
  pltpu.sync_copy(src_hbm.at[pl.ds(base, EPT)], sidx)
  plsc.subcore_barrier()

  def start(j, b):
    pltpu.async_copy(dst_hbm.at[pl.ds(base + j * CK, CK)], didx[b], dsems[b])
    pltpu.async_copy(table.at[sidx.at[pl.ds(j * CK, CK)]], rows[b], gsems[b])

  def finish(j, b):
    pltpu.make_async_copy(dst_hbm.at[pl.ds(base, CK)], didx[b],
                          dsems[b]).wait()
    pltpu.make_async_copy(table.at[sidx.at[pl.ds(j * CK, CK)]], rows[b],
                          gsems[b]).wait()
    pltpu.sync_copy(rows[b], acc.at[didx[b]], add=True)

  for b in range(_M_CS):
    start(b, b)

  def body(t, carry):
    for b in range(_M_CS):
      j = t * _M_CS + b
      finish(j, b)

      @pl.when(j + _M_CS < NCH2)
      def _():
        start(j + _M_CS, b)

    return carry

  lax.fori_loop(0, NCH2 // _M_CS, body, 0)
  plsc.subcore_barrier()
  pltpu.sync_copy(acc.at[pl.ds(s * RPT, RPT)],
                  out_hbm.at[pl.ds(s * RPT, RPT), pl.ds(col0, WH)])


_M_CNT = 8
_K_CNT = 4


@functools.partial(
    pl.kernel,
    out_type=jax.ShapeDtypeStruct((NC, NP, WS), _f32),
    mesh=_mesh,
    scratch_types=[
        pltpu.VMEM((CK, WS), _f32),
        [pltpu.VMEM((CK,), jnp.int32)] * _M_CNT,
        [pltpu.SemaphoreType.DMA] * _M_CNT,
        pltpu.VMEM_SHARED((NP, WS), _f32),
    ],
    compiler_params=pltpu.CompilerParams(use_tc_tiling_on_sc=False),
)
def _sc_counts(dst_hbm, ones_hbm, zero_hbm, out_hbm,
               ones_v, didx, dsems, acc):
  c = lax.axis_index("c")
  s = lax.axis_index("s")
  pltpu.sync_copy(zero_hbm, acc.at[pl.ds(s * RPT, RPT)])
  pltpu.sync_copy(ones_hbm, ones_v)
  base = (s * NC + c) * EPW
  plsc.subcore_barrier()

  def start(j, b):
    pltpu.async_copy(dst_hbm.at[pl.ds(base + j * CK, CK)], didx[b], dsems[b])

  def finish(b):
    pltpu.make_async_copy(dst_hbm.at[pl.ds(base, CK)], didx[b], dsems[b]).wait()
    pltpu.sync_copy(ones_v, acc.at[didx[b]], add=True)

  for b in range(_M_CNT):
    start(b, b)

  main_iters = NCHUNK // _M_CNT

  def body(t, carry):
    for b in range(_M_CNT):
      j = t * _M_CNT + b
      finish(b)

      @pl.when(j + _M_CNT < NCHUNK)
      def _():
        start(j + _M_CNT, b)

    return carry

  lax.fori_loop(0, main_iters, body, 0)
  for j in range(main_iters * _M_CNT, NCHUNK):
    finish(j % _M_CNT)
  plsc.subcore_barrier()
  pltpu.sync_copy(acc.at[pl.ds(s * RPT, RPT)],
                  out_hbm.at[c, pl.ds(s * RPT, RPT)])


_sc_agg_wide = _make_sc_agg(H, m=3, tiled=True)
_sc_agg_narrow = _make_sc_agg(WS, m=6, tiled=False)

_R = 2000
_G = N // _R


def _row_spec(w):
  return pl.BlockSpec((_R, w), lambda i: (i, 0))


def _pair_spec(w):
  return pl.BlockSpec((NC, _R, w), lambda i: (0, i, 0))


def _full_spec(shape):
  nd = len(shape)
  return pl.BlockSpec(shape, lambda i: (0,) * nd)


def _split_body(ei_ref, src_ref, dst_ref):
  src_ref[...] = ei_ref[0]
  dst_ref[...] = ei_ref[1]


def _tc_split_edges(ei):
  return pl.pallas_call(
      _split_body,
      out_shape=[jax.ShapeDtypeStruct((E,), jnp.int32),
                 jax.ShapeDtypeStruct((E,), jnp.int32)],
  )(ei)


def _dual_mm_body(x_ref, wl_ref, wr_ref, br_ref, zl_ref, zr_ref):
  xb = x_ref[...]
  zl_ref[...] = jnp.dot(xb, wl_ref[...], preferred_element_type=_f32)
  zr_ref[...] = jnp.dot(xb, wr_ref[...], preferred_element_type=_f32) + br_ref[...]


def _tc_dual_mm(x, wlT, wrT, br):
  wl_w, wr_w = wlT.shape[1], wrT.shape[1]
  return pl.pallas_call(
      _dual_mm_body,
      grid=(_G,),
      in_specs=[_row_spec(D), _full_spec(wlT.shape), _full_spec(wrT.shape),
                _full_spec(br.shape)],
      out_specs=[_row_spec(wl_w), _row_spec(wr_w)],
      out_shape=[jax.ShapeDtypeStruct((N, wl_w), _f32),
                 jax.ShapeDtypeStruct((N, wr_w), _f32)],
  )(x, wlT, wrT, br)


def _tc_combine1(p, q, zr, wlT, wrT, br):
  def body(p_ref, q_ref, zr_ref, wl_ref, wr_ref, br_ref,
           zl_ref, zro_ref, q_ref_out):
    qsum = q_ref[0] + q_ref[1]
    q_ref_out[...] = qsum
    cnt = jnp.maximum(qsum[:, 0:1], 1.0)
    h = p_ref[...] / cnt + zr_ref[...]
    h = jnp.maximum(h, 0.0)
    zl_ref[...] = jnp.dot(h, wl_ref[...], preferred_element_type=_f32)
    zro_ref[...] = jnp.dot(h, wr_ref[...], preferred_element_type=_f32) + br_ref[...]

  return pl.pallas_call(
      body,
      grid=(_G,),
      in_specs=[_row_spec(H), _pair_spec(WS), _row_spec(H),
                _full_spec(wlT.shape), _full_spec(wrT.shape),
                _full_spec(br.shape)],
      out_specs=[_row_spec(H), _row_spec(H), _row_spec(WS)],
      out_shape=[jax.ShapeDtypeStruct((N, H), _f32),
                 jax.ShapeDtypeStruct((N, H), _f32),
                 jax.ShapeDtypeStruct((N, WS), _f32)],
  )(p, q, zr, wlT, wrT, br)


def _tc_combine2(p, cnt16, zr, wlT, wrT, br):
  def body(p_ref, q_ref, zr_ref, wl_ref, wr_ref, br_ref,
           h_ref, zl_ref, zro_ref):
    cnt = jnp.maximum(q_ref[:, 0:1], 1.0)
    h = p_ref[...] / cnt + zr_ref[...]
    h_ref[...] = h
    zl_ref[...] = jnp.dot(h, wl_ref[...], preferred_element_type=_f32)
    zro_ref[...] = jnp.dot(h, wr_ref[...], preferred_element_type=_f32) + br_ref[...]

  return pl.pallas_call(
      body,
      grid=(_G,),
      in_specs=[_row_spec(H), _row_spec(WS), _row_spec(H),
                _full_spec(wlT.shape), _full_spec(wrT.shape),
                _full_spec(br.shape)],
      out_specs=[_row_spec(H), _row_spec(WS), _row_spec(WS)],
      out_shape=[jax.ShapeDtypeStruct((N, H), _f32),
                 jax.ShapeDtypeStruct((N, WS), _f32),
                 jax.ShapeDtypeStruct((N, WS), _f32)],
  )(p, cnt16, zr, wlT, wrT, br)


def _final_body(p_ref, q_ref, zr_ref, out_ref):
  cnt = jnp.maximum(q_ref[:, 0:1], 1.0)
  out_ref[...] = (p_ref[0] + p_ref[1]) / cnt + zr_ref[...]


def _tc_final(p, cnt16, zr):
  return pl.pallas_call(
      _final_body,
      grid=(_G,),
      in_specs=[_pair_spec(WS), _row_spec(WS), _row_spec(WS)],
      out_specs=_row_spec(WS),
      out_shape=jax.ShapeDtypeStruct((N, WS), _f32),
  )(p, cnt16, zr)


def kernel(x, W0l, b0, W0r, W1l, b1, W1r, W2l, b2, W2r, edge_index):
  def padT(w, width):
    wT = w.T.astype(_f32)
    return jnp.pad(wT, ((0, 0), (0, width - wT.shape[1])))

  wl0T = W0l.T.astype(_f32)
  wr0T = W0r.T.astype(_f32)
  wl1T = W1l.T.astype(_f32)
  wr1T = W1r.T.astype(_f32)
  wl2T = padT(W2l, WS)
  wr2T = padT(W2r, WS)
  br2 = jnp.pad(b2.astype(_f32), (0, WS - C)).reshape(1, WS)

  zeros_cs = jnp.zeros((RPT, WH), _f32)
  zeros_narrow = jnp.zeros((RPT, WS), _f32)
  ones_rows = jnp.ones((CK, WS), _f32)

  src, dst = _tc_split_edges(edge_index)
  q = _sc_counts(dst, ones_rows, zeros_narrow)
  zl0, zr0 = _tc_dual_mm(x, wl0T, wr0T, b0.reshape(1, H))
  p0 = _sc_agg_colsplit(zl0, src, dst, zeros_cs)
  zl1, zr1, cnt16 = _tc_combine1(p0, q, zr0, wl1T, wr1T, b1.reshape(1, H))
  p1 = _sc_agg_colsplit(zl1, src, dst, zeros_cs)
  h1, zl2, zr2 = _tc_combine2(p1, cnt16, zr1, wl2T, wr2T, br2)
  p2 = _sc_agg_narrow(zl2, src, dst, zeros_narrow)
  out = _tc_final(p2, cnt16, zr2)[:, :C]
  return (out, out, h1)


# speedup vs baseline: 1.2287x; 1.2287x over previous
import functools

import jax
import jax.numpy as jnp
from jax import lax
from jax.experimental import pallas as pl
from jax.experimental.pallas import tpu as pltpu
from jax.experimental.pallas import tpu_sc as plsc

N = 10000
E = 320000
D = 128
H = 128
C = 4
NP = 10240

WS = 16

NC = 2
NS = 16
NW = NC * NS
EPW = E // NW
CK = 80
NCHUNK = EPW // CK
RPT = NP // NS

_f32 = jnp.float32

_mesh = plsc.VectorSubcoreMesh(core_axis_name="c", subcore_axis_name="s",
                               num_cores=NC, num_subcores=NS)


def _make_sc_agg(W, m, tiled):
  @functools.partial(
      pl.kernel,
      out_type=jax.ShapeDtypeStruct((NC, NP, W), _f32),
      mesh=_mesh,
      scratch_types=[
          pltpu.VMEM((EPW,), jnp.int32),
          [pltpu.VMEM((CK,), jnp.int32)] * m,
          [pltpu.VMEM((CK, W), _f32)] * m,
          [pltpu.SemaphoreType.DMA] * m,
          [pltpu.SemaphoreType.DMA] * m,
          pltpu.VMEM_SHARED((NP, W), _f32),
      ],
      compiler_params=pltpu.CompilerParams(use_tc_tiling_on_sc=tiled),
  )
  def agg(z_hbm, src_hbm, dst_hbm, zero_hbm, out_hbm,
          sidx, didx, rows, gsems, dsems, acc):
    c = lax.axis_index("c")
    s = lax.axis_index("s")
    pltpu.sync_copy(zero_hbm, acc.at[pl.ds(s * RPT, RPT)])
    base = (s * NC + c) * EPW
    pltpu.sync_copy(src_hbm.at[pl.ds(base, EPW)], sidx)
    plsc.subcore_barrier()

    def start(j, b):
      pltpu.async_copy(dst_hbm.at[pl.ds(base + j * CK, CK)], didx[b], dsems[b])
      pltpu.async_copy(z_hbm.at[sidx.at[pl.ds(j * CK, CK)]], rows[b], gsems[b])

    def finish(j, b):
      pltpu.make_async_copy(dst_hbm.at[pl.ds(base, CK)], didx[b],
                            dsems[b]).wait()
      pltpu.make_async_copy(z_hbm.at[sidx.at[pl.ds(j * CK, CK)]], rows[b],
                            gsems[b]).wait()
      pltpu.sync_copy(rows[b], acc.at[didx[b]], add=True)

    for b in range(m):
      start(b, b)

    main_iters = NCHUNK // m

    def body(t, carry):
      for b in range(m):
        j = t * m + b
        finish(j, b)

        @pl.when(j + m < NCHUNK)
        def _():
          start(j + m, b)

      return carry

    lax.fori_loop(0, main_iters, body, 0)
    for j in range(main_iters * m, NCHUNK):
      finish(j, j % m)
    plsc.subcore_barrier()
    pltpu.sync_copy(acc.at[pl.ds(s * RPT, RPT)],
                    out_hbm.at[c, pl.ds(s * RPT, RPT)])

  return agg


_M_CNT = 8
_K_CNT = 4


@functools.partial(
    pl.kernel,
    out_type=jax.ShapeDtypeStruct((NC, NP, WS), _f32),
    mesh=_mesh,
    scratch_types=[
        pltpu.VMEM((CK, WS), _f32),
        [pltpu.VMEM((CK,), jnp.int32)] * _M_CNT,
        [pltpu.SemaphoreType.DMA] * _M_CNT,
        pltpu.VMEM_SHARED((NP, WS), _f32),
    ],
    compiler_params=pltpu.CompilerParams(use_tc_tiling_on_sc=False),
)
def _sc_counts(dst_hbm, ones_hbm, zero_hbm, out_hbm,
               ones_v, didx, dsems, acc):
  c = lax.axis_index("c")
  s = lax.axis_index("s")
  pltpu.sync_copy(zero_hbm, acc.at[pl.ds(s * RPT, RPT)])
  pltpu.sync_copy(ones_hbm, ones_v)
  base = (s * NC + c) * EPW
  plsc.subcore_barrier()

  def start(j, b):
    pltpu.async_copy(dst_hbm.at[pl.ds(base + j * CK, CK)], didx[b], dsems[b])

  def finish(b):
    pltpu.make_async_copy(dst_hbm.at[pl.ds(base, CK)], didx[b], dsems[b]).wait()
    pltpu.sync_copy(ones_v, acc.at[didx[b]], add=True)

  for b in range(_M_CNT):
    start(b, b)

  main_iters = NCHUNK // _M_CNT

  def body(t, carry):
    for b in range(_M_CNT):
      j = t * _M_CNT + b
      finish(b)

      @pl.when(j + _M_CNT < NCHUNK)
      def _():
        start(j + _M_CNT, b)

    return carry

  lax.fori_loop(0, main_iters, body, 0)
  for j in range(main_iters * _M_CNT, NCHUNK):
    finish(j % _M_CNT)
  plsc.subcore_barrier()
  pltpu.sync_copy(acc.at[pl.ds(s * RPT, RPT)],
                  out_hbm.at[c, pl.ds(s * RPT, RPT)])


_sc_agg_wide = _make_sc_agg(H, m=3, tiled=True)
_sc_agg_narrow = _make_sc_agg(WS, m=6, tiled=False)

_R = 2000
_G = N // _R


def _row_spec(w):
  return pl.BlockSpec((_R, w), lambda i: (i, 0))


def _pair_spec(w):
  return pl.BlockSpec((NC, _R, w), lambda i: (0, i, 0))


def _full_spec(shape):
  nd = len(shape)
  return pl.BlockSpec(shape, lambda i: (0,) * nd)


def _split_body(ei_ref, src_ref, dst_ref):
  src_ref[...] = ei_ref[0]
  dst_ref[...] = ei_ref[1]


def _tc_split_edges(ei):
  return pl.pallas_call(
      _split_body,
      out_shape=[jax.ShapeDtypeStruct((E,), jnp.int32),
                 jax.ShapeDtypeStruct((E,), jnp.int32)],
  )(ei)


def _dual_mm_body(x_ref, wl_ref, wr_ref, br_ref, zl_ref, zr_ref):
  xb = x_ref[...]
  zl_ref[...] = jnp.dot(xb, wl_ref[...], preferred_element_type=_f32)
  zr_ref[...] = jnp.dot(xb, wr_ref[...], preferred_element_type=_f32) + br_ref[...]


def _tc_dual_mm(x, wlT, wrT, br):
  wl_w, wr_w = wlT.shape[1], wrT.shape[1]
  return pl.pallas_call(
      _dual_mm_body,
      grid=(_G,),
      in_specs=[_row_spec(D), _full_spec(wlT.shape), _full_spec(wrT.shape),
                _full_spec(br.shape)],
      out_specs=[_row_spec(wl_w), _row_spec(wr_w)],
      out_shape=[jax.ShapeDtypeStruct((N, wl_w), _f32),
                 jax.ShapeDtypeStruct((N, wr_w), _f32)],
  )(x, wlT, wrT, br)


def _tc_combine1(p, q, zr, wlT, wrT, br, wl12, wr12, bq, brw):
  def body(p_ref, q_ref, zr_ref, wl_ref, wr_ref, br_ref, wl12_ref, wr12_ref,
           bq_ref, brw_ref, zl_ref, zro_ref, v_ref, w_ref, q_ref_out):
    qsum = q_ref[0] + q_ref[1]
    q_ref_out[...] = qsum
    cnt = jnp.maximum(qsum[:, 0:1], 1.0)
    h = (p_ref[0] + p_ref[1]) / cnt + zr_ref[...]
    h = jnp.maximum(h, 0.0)
    zl_ref[...] = jnp.dot(h, wl_ref[...], preferred_element_type=_f32)
    zro_ref[...] = jnp.dot(h, wr_ref[...], preferred_element_type=_f32) + br_ref[...]
    v_ref[...] = jnp.dot(h, wl12_ref[...], preferred_element_type=_f32) + bq_ref[...]
    w_ref[...] = jnp.dot(h, wr12_ref[...], preferred_element_type=_f32) + brw_ref[...]

  return pl.pallas_call(
      body,
      grid=(_G,),
      in_specs=[_pair_spec(H), _pair_spec(WS), _row_spec(H),
                _full_spec(wlT.shape), _full_spec(wrT.shape),
                _full_spec(br.shape), _full_spec(wl12.shape),
                _full_spec(wr12.shape), _full_spec(bq.shape),
                _full_spec(brw.shape)],
      out_specs=[_row_spec(H), _row_spec(H), _row_spec(WS), _row_spec(WS),
                 _row_spec(WS)],
      out_shape=[jax.ShapeDtypeStruct((N, H), _f32),
                 jax.ShapeDtypeStruct((N, H), _f32),
                 jax.ShapeDtypeStruct((N, WS), _f32),
                 jax.ShapeDtypeStruct((N, WS), _f32),
                 jax.ShapeDtypeStruct((N, WS), _f32)],
  )(p, q, zr, wlT, wrT, br, wl12, wr12, bq, brw)


def _tc_zl2(p, cnt16, zq, wl2T):
  def body(p_ref, q_ref, zq_ref, wl2_ref, out_ref):
    cnt = jnp.maximum(q_ref[:, 0:1], 1.0)
    m1 = (p_ref[0] + p_ref[1]) / cnt
    out_ref[...] = (jnp.dot(m1, wl2_ref[...], preferred_element_type=_f32)
                    + zq_ref[...])

  return pl.pallas_call(
      body,
      grid=(_G,),
      in_specs=[_pair_spec(H), _row_spec(WS), _row_spec(WS),
                _full_spec(wl2T.shape)],
      out_specs=_row_spec(WS),
      out_shape=jax.ShapeDtypeStruct((N, WS), _f32),
  )(p, cnt16, zq, wl2T)


def _tc_h1(p, cnt16, zr):
  def body(p_ref, q_ref, zr_ref, h_ref):
    cnt = jnp.maximum(q_ref[:, 0:1], 1.0)
    h_ref[...] = (p_ref[0] + p_ref[1]) / cnt + zr_ref[...]

  return pl.pallas_call(
      body,
      grid=(_G,),
      in_specs=[_pair_spec(H), _row_spec(WS), _row_spec(H)],
      out_specs=_row_spec(H),
      out_shape=jax.ShapeDtypeStruct((N, H), _f32),
  )(p, cnt16, zr)


def _tc_final(p1, p2, cnt16, w, wr2T):
  def body(p1_ref, p2_ref, q_ref, w_ref, wr2_ref, out_ref):
    cnt = jnp.maximum(q_ref[:, 0:1], 1.0)
    m1 = (p1_ref[0] + p1_ref[1]) / cnt
    out_ref[...] = (jnp.dot(m1, wr2_ref[...], preferred_element_type=_f32)
                    + (p2_ref[0] + p2_ref[1]) / cnt + w_ref[...])

  return pl.pallas_call(
      body,
      grid=(_G,),
      in_specs=[_pair_spec(H), _pair_spec(WS), _row_spec(WS), _row_spec(WS),
                _full_spec(wr2T.shape)],
      out_specs=_row_spec(WS),
      out_shape=jax.ShapeDtypeStruct((N, WS), _f32),
  )(p1, p2, cnt16, w, wr2T)


def kernel(x, W0l, b0, W0r, W1l, b1, W1r, W2l, b2, W2r, edge_index):
  def padT(w, width):
    wT = w.T.astype(_f32)
    return jnp.pad(wT, ((0, 0), (0, width - wT.shape[1])))

  wl0T = W0l.T.astype(_f32)
  wr0T = W0r.T.astype(_f32)
  wl1T = W1l.T.astype(_f32)
  wr1T = W1r.T.astype(_f32)
  wl2T = padT(W2l, WS)
  wr2T = padT(W2r, WS)
  br2 = jnp.pad(b2.astype(_f32), (0, WS - C)).reshape(1, WS)
  wl12 = wr1T @ wl2T
  wr12 = wr1T @ wr2T
  bq = b1.astype(_f32).reshape(1, H) @ wl2T
  brw = b1.astype(_f32).reshape(1, H) @ wr2T + br2

  zeros_wide = jnp.zeros((RPT, H), _f32)
  zeros_narrow = jnp.zeros((RPT, WS), _f32)
  ones_rows = jnp.ones((CK, WS), _f32)

  src, dst = _tc_split_edges(edge_index)
  q = _sc_counts(dst, ones_rows, zeros_narrow)
  zl0, zr0 = _tc_dual_mm(x, wl0T, wr0T, b0.reshape(1, H))
  p0 = _sc_agg_wide(zl0, src, dst, zeros_wide)
  zl1, zr1, zq, w, cnt16 = _tc_combine1(p0, q, zr0, wl1T, wr1T,
                                        b1.reshape(1, H), wl12, wr12, bq, brw)
  p1 = _sc_agg_wide(zl1, src, dst, zeros_wide)
  zl2 = _tc_zl2(p1, cnt16, zq, wl2T)
  p2 = _sc_agg_narrow(zl2, src, dst, zeros_narrow)
  h1 = _tc_h1(p1, cnt16, zr1)
  out = _tc_final(p1, p2, cnt16, w, wr2T)[:, :C]
  return (out, out, h1)


# speedup vs baseline: 1.2585x; 1.0243x over previous
import functools

import jax
import jax.numpy as jnp
from jax import lax
from jax.experimental import pallas as pl
from jax.experimental.pallas import tpu as pltpu
from jax.experimental.pallas import tpu_sc as plsc

N = 10000
E = 320000
D = 128
H = 128
C = 4
NP = 10240

WS = 16

NC = 2
NS = 16
NW = NC * NS
EPW = E // NW
CK = 80
NCHUNK = EPW // CK
RPT = NP // NS

_f32 = jnp.float32

_mesh = plsc.VectorSubcoreMesh(core_axis_name="c", subcore_axis_name="s",
                               num_cores=NC, num_subcores=NS)


def _make_sc_agg(W, m, tiled):
  @functools.partial(
      pl.kernel,
      out_type=jax.ShapeDtypeStruct((NC, NP, W), _f32),
      mesh=_mesh,
      scratch_types=[
          pltpu.VMEM((EPW,), jnp.int32),
          [pltpu.VMEM((CK,), jnp.int32)] * m,
          [pltpu.VMEM((CK, W), _f32)] * m,
          [pltpu.SemaphoreType.DMA] * m,
          [pltpu.SemaphoreType.DMA] * m,
          pltpu.VMEM_SHARED((NP, W), _f32),
      ],
      compiler_params=pltpu.CompilerParams(use_tc_tiling_on_sc=tiled),
  )
  def agg(z_hbm, src_hbm, dst_hbm, zero_hbm, out_hbm,
          sidx, didx, rows, gsems, dsems, acc):
    c = lax.axis_index("c")
    s = lax.axis_index("s")
    pltpu.sync_copy(zero_hbm, acc.at[pl.ds(s * RPT, RPT)])
    base = (s * NC + c) * EPW
    pltpu.sync_copy(src_hbm.at[pl.ds(base, EPW)], sidx)
    plsc.subcore_barrier()

    def start(j, b):
      pltpu.async_copy(dst_hbm.at[pl.ds(base + j * CK, CK)], didx[b], dsems[b])
      pltpu.async_copy(z_hbm.at[sidx.at[pl.ds(j * CK, CK)]], rows[b], gsems[b])

    def finish(j, b):
      pltpu.make_async_copy(dst_hbm.at[pl.ds(base, CK)], didx[b],
                            dsems[b]).wait()
      pltpu.make_async_copy(z_hbm.at[sidx.at[pl.ds(j * CK, CK)]], rows[b],
                            gsems[b]).wait()
      pltpu.sync_copy(rows[b], acc.at[didx[b]], add=True)

    for b in range(m):
      start(b, b)

    main_iters = NCHUNK // m

    def body(t, carry):
      for b in range(m):
        j = t * m + b
        finish(j, b)

        @pl.when(j + m < NCHUNK)
        def _():
          start(j + m, b)

      return carry

    lax.fori_loop(0, main_iters, body, 0)
    for j in range(main_iters * m, NCHUNK):
      finish(j, j % m)
    plsc.subcore_barrier()
    pltpu.sync_copy(acc.at[pl.ds(s * RPT, RPT)],
                    out_hbm.at[c, pl.ds(s * RPT, RPT)])

  return agg


_M_CNT = 8
_K_CNT = 4


@functools.partial(
    pl.kernel,
    out_type=jax.ShapeDtypeStruct((NC, NP, WS), _f32),
    mesh=_mesh,
    scratch_types=[
        pltpu.VMEM((CK, WS), _f32),
        [pltpu.VMEM((CK,), jnp.int32)] * _M_CNT,
        [pltpu.SemaphoreType.DMA] * _M_CNT,
        pltpu.VMEM_SHARED((NP, WS), _f32),
    ],
    compiler_params=pltpu.CompilerParams(use_tc_tiling_on_sc=False),
)
def _sc_counts(dst_hbm, ones_hbm, zero_hbm, out_hbm,
               ones_v, didx, dsems, acc):
  c = lax.axis_index("c")
  s = lax.axis_index("s")
  pltpu.sync_copy(zero_hbm, acc.at[pl.ds(s * RPT, RPT)])
  pltpu.sync_copy(ones_hbm, ones_v)
  base = (s * NC + c) * EPW
  plsc.subcore_barrier()

  def start(j, b):
    pltpu.async_copy(dst_hbm.at[pl.ds(base + j * CK, CK)], didx[b], dsems[b])

  def finish(b):
    pltpu.make_async_copy(dst_hbm.at[pl.ds(base, CK)], didx[b], dsems[b]).wait()
    pltpu.sync_copy(ones_v, acc.at[didx[b]], add=True)

  for b in range(_M_CNT):
    start(b, b)

  main_iters = NCHUNK // _M_CNT

  def body(t, carry):
    for b in range(_M_CNT):
      j = t * _M_CNT + b
      finish(b)

      @pl.when(j + _M_CNT < NCHUNK)
      def _():
        start(j + _M_CNT, b)

    return carry

  lax.fori_loop(0, main_iters, body, 0)
  for j in range(main_iters * _M_CNT, NCHUNK):
    finish(j % _M_CNT)
  plsc.subcore_barrier()
  pltpu.sync_copy(acc.at[pl.ds(s * RPT, RPT)],
                  out_hbm.at[c, pl.ds(s * RPT, RPT)])


_sc_agg_wide = _make_sc_agg(H, m=3, tiled=True)
_sc_agg_narrow = _make_sc_agg(WS, m=8, tiled=False)

_R = 2000
_G = N // _R


def _row_spec(w):
  return pl.BlockSpec((_R, w), lambda i: (i, 0))


def _pair_spec(w):
  return pl.BlockSpec((NC, _R, w), lambda i: (0, i, 0))


def _full_spec(shape):
  nd = len(shape)
  return pl.BlockSpec(shape, lambda i: (0,) * nd)


def _split_body(ei_ref, src_ref, dst_ref):
  src_ref[...] = ei_ref[0]
  dst_ref[...] = ei_ref[1]


def _tc_split_edges(ei):
  return pl.pallas_call(
      _split_body,
      out_shape=[jax.ShapeDtypeStruct((E,), jnp.int32),
                 jax.ShapeDtypeStruct((E,), jnp.int32)],
  )(ei)


def _dual_mm_body(x_ref, wl_ref, wr_ref, br_ref, zl_ref, zr_ref):
  xb = x_ref[...]
  zl_ref[...] = jnp.dot(xb, wl_ref[...], preferred_element_type=_f32)
  zr_ref[...] = jnp.dot(xb, wr_ref[...], preferred_element_type=_f32) + br_ref[...]


def _tc_dual_mm(x, wlT, wrT, br):
  wl_w, wr_w = wlT.shape[1], wrT.shape[1]
  return pl.pallas_call(
      _dual_mm_body,
      grid=(_G,),
      in_specs=[_row_spec(D), _full_spec(wlT.shape), _full_spec(wrT.shape),
                _full_spec(br.shape)],
      out_specs=[_row_spec(wl_w), _row_spec(wr_w)],
      out_shape=[jax.ShapeDtypeStruct((N, wl_w), _f32),
                 jax.ShapeDtypeStruct((N, wr_w), _f32)],
  )(x, wlT, wrT, br)


def _tc_combine1(p, q, zr, wlT, wrT, br):
  def body(p_ref, q_ref, zr_ref, wl_ref, wr_ref, br_ref,
           zl_ref, zro_ref, q_ref_out):
    qsum = q_ref[0] + q_ref[1]
    q_ref_out[...] = qsum
    cnt = jnp.maximum(qsum[:, 0:1], 1.0)
    h = (p_ref[0] + p_ref[1]) / cnt + zr_ref[...]
    h = jnp.maximum(h, 0.0)
    zl_ref[...] = jnp.dot(h, wl_ref[...], preferred_element_type=_f32)
    zro_ref[...] = jnp.dot(h, wr_ref[...], preferred_element_type=_f32) + br_ref[...]

  return pl.pallas_call(
      body,
      grid=(_G,),
      in_specs=[_pair_spec(H), _pair_spec(WS), _row_spec(H),
                _full_spec(wlT.shape), _full_spec(wrT.shape),
                _full_spec(br.shape)],
      out_specs=[_row_spec(H), _row_spec(H), _row_spec(WS)],
      out_shape=[jax.ShapeDtypeStruct((N, H), _f32),
                 jax.ShapeDtypeStruct((N, H), _f32),
                 jax.ShapeDtypeStruct((N, WS), _f32)],
  )(p, q, zr, wlT, wrT, br)


def _tc_combine2(p, cnt16, zr, wlT, wrT, br):
  def body(p_ref, q_ref, zr_ref, wl_ref, wr_ref, br_ref,
           h_ref, zl_ref, zro_ref):
    cnt = jnp.maximum(q_ref[:, 0:1], 1.0)
    h = (p_ref[0] + p_ref[1]) / cnt + zr_ref[...]
    h_ref[...] = h
    zl_ref[...] = jnp.dot(h, wl_ref[...], preferred_element_type=_f32)
    zro_ref[...] = jnp.dot(h, wr_ref[...], preferred_element_type=_f32) + br_ref[...]

  return pl.pallas_call(
      body,
      grid=(_G,),
      in_specs=[_pair_spec(H), _row_spec(WS), _row_spec(H),
                _full_spec(wlT.shape), _full_spec(wrT.shape),
                _full_spec(br.shape)],
      out_specs=[_row_spec(H), _row_spec(WS), _row_spec(WS)],
      out_shape=[jax.ShapeDtypeStruct((N, H), _f32),
                 jax.ShapeDtypeStruct((N, WS), _f32),
                 jax.ShapeDtypeStruct((N, WS), _f32)],
  )(p, cnt16, zr, wlT, wrT, br)


def _final_body(p_ref, q_ref, zr_ref, out_ref):
  cnt = jnp.maximum(q_ref[:, 0:1], 1.0)
  out_ref[...] = (p_ref[0] + p_ref[1]) / cnt + zr_ref[...]


def _tc_final(p, cnt16, zr):
  return pl.pallas_call(
      _final_body,
      grid=(_G,),
      in_specs=[_pair_spec(WS), _row_spec(WS), _row_spec(WS)],
      out_specs=_row_spec(WS),
      out_shape=jax.ShapeDtypeStruct((N, WS), _f32),
  )(p, cnt16, zr)


def kernel(x, W0l, b0, W0r, W1l, b1, W1r, W2l, b2, W2r, edge_index):
  def padT(w, width):
    wT = w.T.astype(_f32)
    return jnp.pad(wT, ((0, 0), (0, width - wT.shape[1])))

  wl0T = W0l.T.astype(_f32)
  wr0T = W0r.T.astype(_f32)
  wl1T = W1l.T.astype(_f32)
  wr1T = W1r.T.astype(_f32)
  wl2T = padT(W2l, WS)
  wr2T = padT(W2r, WS)
  br2 = jnp.pad(b2.astype(_f32), (0, WS - C)).reshape(1, WS)

  zeros_wide = jnp.zeros((RPT, H), _f32)
  zeros_narrow = jnp.zeros((RPT, WS), _f32)
  ones_rows = jnp.ones((CK, WS), _f32)

  src, dst = _tc_split_edges(edge_index)
  q = _sc_counts(dst, ones_rows, zeros_narrow)
  zl0, zr0 = _tc_dual_mm(x, wl0T, wr0T, b0.reshape(1, H))
  p0 = _sc_agg_wide(zl0, src, dst, zeros_wide)
  zl1, zr1, cnt16 = _tc_combine1(p0, q, zr0, wl1T, wr1T, b1.reshape(1, H))
  p1 = _sc_agg_wide(zl1, src, dst, zeros_wide)
  h1, zl2, zr2 = _tc_combine2(p1, cnt16, zr1, wl2T, wr2T, br2)
  p2 = _sc_agg_narrow(zl2, src, dst, zeros_narrow)
  out = _tc_final(p2, cnt16, zr2)[:, :C]
  return (out, out, h1)


# speedup vs baseline: 1.2611x; 1.0021x over previous
import functools

import jax
import jax.numpy as jnp
from jax import lax
from jax.experimental import pallas as pl
from jax.experimental.pallas import tpu as pltpu
from jax.experimental.pallas import tpu_sc as plsc

N = 10000
E = 320000
D = 128
H = 128
C = 4
NP = 10240

WS = 16

NC = 2
NS = 16
NW = NC * NS
EPW = E // NW
CK = 80
NCHUNK = EPW // CK
RPT = NP // NS

_f32 = jnp.float32

_mesh = plsc.VectorSubcoreMesh(core_axis_name="c", subcore_axis_name="s",
                               num_cores=NC, num_subcores=NS)


def _make_sc_agg(W, m, tiled):
  @functools.partial(
      pl.kernel,
      out_type=jax.ShapeDtypeStruct((NC, NP, W), _f32),
      mesh=_mesh,
      scratch_types=[
          pltpu.VMEM((EPW,), jnp.int32),
          [pltpu.VMEM((CK,), jnp.int32)] * m,
          [pltpu.VMEM((CK, W), _f32)] * m,
          [pltpu.SemaphoreType.DMA] * m,
          [pltpu.SemaphoreType.DMA] * m,
          pltpu.VMEM_SHARED((NP, W), _f32),
      ],
      compiler_params=pltpu.CompilerParams(use_tc_tiling_on_sc=tiled),
  )
  def agg(z_hbm, src_hbm, dst_hbm, zero_hbm, out_hbm,
          sidx, didx, rows, gsems, dsems, acc):
    c = lax.axis_index("c")
    s = lax.axis_index("s")
    pltpu.sync_copy(zero_hbm, acc.at[pl.ds(s * RPT, RPT)])
    base = (s * NC + c) * EPW
    pltpu.sync_copy(src_hbm.at[pl.ds(base, EPW)], sidx)
    plsc.subcore_barrier()

    def start(j, b):
      pltpu.async_copy(dst_hbm.at[pl.ds(base + j * CK, CK)], didx[b], dsems[b])
      pltpu.async_copy(z_hbm.at[sidx.at[pl.ds(j * CK, CK)]], rows[b], gsems[b])

    def finish(j, b):
      pltpu.make_async_copy(dst_hbm.at[pl.ds(base, CK)], didx[b],
                            dsems[b]).wait()
      pltpu.make_async_copy(z_hbm.at[sidx.at[pl.ds(j * CK, CK)]], rows[b],
                            gsems[b]).wait()
      pltpu.sync_copy(rows[b], acc.at[didx[b]], add=True)

    for b in range(m):
      start(b, b)

    main_iters = NCHUNK // m

    def body(t, carry):
      for b in range(m):
        j = t * m + b
        finish(j, b)

        @pl.when(j + m < NCHUNK)
        def _():
          start(j + m, b)

      return carry

    lax.fori_loop(0, main_iters, body, 0)
    for j in range(main_iters * m, NCHUNK):
      finish(j, j % m)
    plsc.subcore_barrier()
    pltpu.sync_copy(acc.at[pl.ds(s * RPT, RPT)],
                    out_hbm.at[c, pl.ds(s * RPT, RPT)])

  return agg


_M_CNT = 8
_K_CNT = 4


@functools.partial(
    pl.kernel,
    out_type=jax.ShapeDtypeStruct((NC, NP, WS), _f32),
    mesh=_mesh,
    scratch_types=[
        pltpu.VMEM((CK, WS), _f32),
        [pltpu.VMEM((CK,), jnp.int32)] * _M_CNT,
        [pltpu.SemaphoreType.DMA] * _M_CNT,
        pltpu.VMEM_SHARED((NP, WS), _f32),
    ],
    compiler_params=pltpu.CompilerParams(use_tc_tiling_on_sc=False),
)
def _sc_counts(dst_hbm, ones_hbm, zero_hbm, out_hbm,
               ones_v, didx, dsems, acc):
  c = lax.axis_index("c")
  s = lax.axis_index("s")
  pltpu.sync_copy(zero_hbm, acc.at[pl.ds(s * RPT, RPT)])
  pltpu.sync_copy(ones_hbm, ones_v)
  base = (s * NC + c) * EPW
  plsc.subcore_barrier()

  def start(j, b):
    pltpu.async_copy(dst_hbm.at[pl.ds(base + j * CK, CK)], didx[b], dsems[b])

  def finish(b):
    pltpu.make_async_copy(dst_hbm.at[pl.ds(base, CK)], didx[b], dsems[b]).wait()
    pltpu.sync_copy(ones_v, acc.at[didx[b]], add=True)

  for b in range(_M_CNT):
    start(b, b)

  main_iters = NCHUNK // _M_CNT

  def body(t, carry):
    for b in range(_M_CNT):
      j = t * _M_CNT + b
      finish(b)

      @pl.when(j + _M_CNT < NCHUNK)
      def _():
        start(j + _M_CNT, b)

    return carry

  lax.fori_loop(0, main_iters, body, 0)
  for j in range(main_iters * _M_CNT, NCHUNK):
    finish(j % _M_CNT)
  plsc.subcore_barrier()
  pltpu.sync_copy(acc.at[pl.ds(s * RPT, RPT)],
                  out_hbm.at[c, pl.ds(s * RPT, RPT)])


_sc_agg_wide = _make_sc_agg(H, m=3, tiled=True)
_sc_agg_narrow = _make_sc_agg(WS, m=6, tiled=False)

_R = 2000
_G = N // _R


def _row_spec(w):
  return pl.BlockSpec((_R, w), lambda i: (i, 0))


def _pair_spec(w):
  return pl.BlockSpec((NC, _R, w), lambda i: (0, i, 0))


def _full_spec(shape):
  nd = len(shape)
  return pl.BlockSpec(shape, lambda i: (0,) * nd)


def _split_body(ei_ref, src_ref, dst_ref):
  src_ref[...] = ei_ref[0]
  dst_ref[...] = ei_ref[1]


def _tc_split_edges(ei):
  return pl.pallas_call(
      _split_body,
      out_shape=[jax.ShapeDtypeStruct((E,), jnp.int32),
                 jax.ShapeDtypeStruct((E,), jnp.int32)],
  )(ei)


def _dual_mm_body(x_ref, wl_ref, wr_ref, br_ref, zl_ref, zr_ref):
  xb = x_ref[...]
  zl_ref[...] = jnp.dot(xb, wl_ref[...], preferred_element_type=_f32)
  zr_ref[...] = jnp.dot(xb, wr_ref[...], preferred_element_type=_f32) + br_ref[...]


def _tc_dual_mm(x, wlT, wrT, br):
  wl_w, wr_w = wlT.shape[1], wrT.shape[1]
  return pl.pallas_call(
      _dual_mm_body,
      grid=(_G,),
      in_specs=[_row_spec(D), _full_spec(wlT.shape), _full_spec(wrT.shape),
                _full_spec(br.shape)],
      out_specs=[_row_spec(wl_w), _row_spec(wr_w)],
      out_shape=[jax.ShapeDtypeStruct((N, wl_w), _f32),
                 jax.ShapeDtypeStruct((N, wr_w), _f32)],
  )(x, wlT, wrT, br)


def _tc_combine1(p, q, zr, wlT, wrT, br):
  def body(p_ref, q_ref, zr_ref, wl_ref, wr_ref, br_ref,
           zl_ref, zro_ref, q_ref_out):
    qsum = q_ref[0] + q_ref[1]
    q_ref_out[...] = qsum
    cnt = jnp.maximum(qsum[:, 0:1], 1.0)
    h = (p_ref[0] + p_ref[1]) / cnt + zr_ref[...]
    h = jnp.maximum(h, 0.0)
    zl_ref[...] = jnp.dot(h, wl_ref[...], preferred_element_type=_f32)
    zro_ref[...] = jnp.dot(h, wr_ref[...], preferred_element_type=_f32) + br_ref[...]

  return pl.pallas_call(
      body,
      grid=(_G,),
      in_specs=[_pair_spec(H), _pair_spec(WS), _row_spec(H),
                _full_spec(wlT.shape), _full_spec(wrT.shape),
                _full_spec(br.shape)],
      out_specs=[_row_spec(H), _row_spec(H), _row_spec(WS)],
      out_shape=[jax.ShapeDtypeStruct((N, H), _f32),
                 jax.ShapeDtypeStruct((N, H), _f32),
                 jax.ShapeDtypeStruct((N, WS), _f32)],
  )(p, q, zr, wlT, wrT, br)


def _tc_combine2(p, cnt16, zr, wlT, wrT, br):
  def body(p_ref, q_ref, zr_ref, wl_ref, wr_ref, br_ref,
           h_ref, zl_ref, zro_ref):
    cnt = jnp.maximum(q_ref[:, 0:1], 1.0)
    h = (p_ref[0] + p_ref[1]) / cnt + zr_ref[...]
    h_ref[...] = h
    zl_ref[...] = jnp.dot(h, wl_ref[...], preferred_element_type=_f32)
    zro_ref[...] = jnp.dot(h, wr_ref[...], preferred_element_type=_f32) + br_ref[...]

  return pl.pallas_call(
      body,
      grid=(_G,),
      in_specs=[_pair_spec(H), _row_spec(WS), _row_spec(H),
                _full_spec(wlT.shape), _full_spec(wrT.shape),
                _full_spec(br.shape)],
      out_specs=[_row_spec(H), _row_spec(WS), _row_spec(WS)],
      out_shape=[jax.ShapeDtypeStruct((N, H), _f32),
                 jax.ShapeDtypeStruct((N, WS), _f32),
                 jax.ShapeDtypeStruct((N, WS), _f32)],
  )(p, cnt16, zr, wlT, wrT, br)


def _final_body(p_ref, q_ref, zr_ref, out_ref):
  cnt = jnp.maximum(q_ref[:, 0:1], 1.0)
  out_ref[...] = (p_ref[0] + p_ref[1]) / cnt + zr_ref[...]


def _tc_final(p, cnt16, zr):
  return pl.pallas_call(
      _final_body,
      grid=(_G,),
      in_specs=[_pair_spec(WS), _row_spec(WS), _row_spec(WS)],
      out_specs=_row_spec(WS),
      out_shape=jax.ShapeDtypeStruct((N, WS), _f32),
  )(p, cnt16, zr)


def kernel(x, W0l, b0, W0r, W1l, b1, W1r, W2l, b2, W2r, edge_index):
  def padT(w, width):
    wT = w.T.astype(_f32)
    return jnp.pad(wT, ((0, 0), (0, width - wT.shape[1])))

  wl0T = W0l.T.astype(_f32)
  wr0T = W0r.T.astype(_f32)
  wl1T = W1l.T.astype(_f32)
  wr1T = W1r.T.astype(_f32)
  wl2T = padT(W2l, WS)
  wr2T = padT(W2r, WS)
  br2 = jnp.pad(b2.astype(_f32), (0, WS - C)).reshape(1, WS)

  zeros_wide = jnp.zeros((RPT, H), _f32)
  zeros_narrow = jnp.zeros((RPT, WS), _f32)
  ones_rows = jnp.ones((CK, WS), _f32)

  src, dst = _tc_split_edges(edge_index)
  q = _sc_counts(dst, ones_rows, zeros_narrow)
  zl0, zr0 = _tc_dual_mm(x, wl0T, wr0T, b0.reshape(1, H))
  p0 = _sc_agg_wide(zl0, src, dst, zeros_wide)
  zl1, zr1, cnt16 = _tc_combine1(p0, q, zr0, wl1T, wr1T, b1.reshape(1, H))
  p1 = _sc_agg_wide(zl1, src, dst, zeros_wide)
  h1, zl2, zr2 = _tc_combine2(p1, cnt16, zr1, wl2T, wr2T, br2)
  p2 = _sc_agg_narrow(zl2, src, dst, zeros_narrow)
  out = _tc_final(p2, cnt16, zr2)[:, :C]
  return (out, out, h1)
